# baseline (speedup 1.0000x reference)
import math

import jax
import jax.numpy as jnp
from jax.experimental import pallas as pl
from jax.experimental.pallas import tpu as pltpu

_B, _T, _DIM = 4, 4096, 1024
_S, _H = 8, 8
_HD = _DIM // _H
_L = _T // _S


def _memory_kernel(x_ref, wcat_ref, msT_ref, wwq_ref, wkv_ref, wm_ref,
                   out_ref, memout_ref, mem_scratch):
    j = pl.program_id(1)
    s_idx = _S - 1 - j

    @pl.when(j == 0)
    def _():
        mem_scratch[...] = jnp.zeros_like(mem_scratch)

    xb = x_ref[0, 0]
    kvq = jnp.dot(xb, wcat_ref[...], preferred_element_type=jnp.float32)
    k2 = kvq[:, :_DIM]
    v2 = kvq[:, _DIM:2 * _DIM]
    qr = kvq[:, 2 * _DIM:]

    qwT = jnp.dot(wwq_ref[...], msT_ref[...], preferred_element_type=jnp.float32)
    col_s = jax.lax.broadcasted_iota(jnp.int32, (1, _S), 1)
    q_col = jnp.sum(jnp.where(col_s == s_idx, qwT, 0.0), axis=1, keepdims=True)
    row_head = jax.lax.broadcasted_iota(jnp.int32, (_DIM, _S), 0) // _HD
    pmask = row_head == jax.lax.broadcasted_iota(jnp.int32, (_DIM, _S), 1)
    qblk = jnp.where(pmask, q_col * (1.0 / math.sqrt(_HD)), 0.0)

    scores = jnp.dot(k2, qblk, preferred_element_type=jnp.float32)
    mw = jnp.max(scores, axis=0, keepdims=True)
    ew = jnp.exp(scores - mw)
    attn_w = ew / jnp.sum(ew, axis=0, keepdims=True)

    mem2 = jax.lax.dot_general(attn_w, v2, (((0,), (0,)), ((), ())),
                               preferred_element_type=jnp.float32)
    dmask = (jax.lax.broadcasted_iota(jnp.int32, (_H, _DIM), 1) // _HD
             == jax.lax.broadcasted_iota(jnp.int32, (_H, _DIM), 0))
    mem_row = jnp.sum(jnp.where(dmask, mem2, 0.0), axis=0, keepdims=True)
    mem_row = mem_row * wm_ref[...]
    var = jnp.sum(mem_row * mem_row, axis=1, keepdims=True) * (1.0 / _DIM)
    mem_row = mem_row * jax.lax.rsqrt(var + 1e-6)

    mem_scratch[pl.ds(s_idx, 1)] = mem_row.reshape(1, 1, _DIM)
    memout_ref[...] = mem_row.reshape(1, 1, 1, _DIM)

    mem_all = mem_scratch[...].reshape(_S, _DIM)
    kvr = jnp.dot(mem_all, wkv_ref[...], preferred_element_type=jnp.float32)
    kr = kvr[:, :_DIM]
    vr = kvr[:, _DIM:]
    qk = jax.lax.dot_general(qr, kr, (((1,), (1,)), ((), ())),
                             preferred_element_type=jnp.float32) * (1.0 / math.sqrt(_DIM))
    valid = col_s >= s_idx
    qk = jnp.where(valid, qk, -1e30)
    mr = jnp.max(qk, axis=1, keepdims=True)
    er = jnp.exp(qk - mr)
    attn_r = er / jnp.sum(er, axis=1, keepdims=True)
    vr = jnp.where(jax.lax.broadcasted_iota(jnp.int32, (_S, 1), 0) >= s_idx, vr, 0.0)
    out = jnp.dot(attn_r, vr, preferred_element_type=jnp.float32)
    out_ref[0, 0] = out * jax.nn.sigmoid(out)


def _run(x, memory_slots, W_read_q, W_read_kv, W_write_q, W_write_k, W_write_v,
         write_matter, interpret=False):
    xr = x.reshape(_B, _S, _L, _DIM)
    wcat = jnp.concatenate([W_write_k.T, W_write_v.T, W_read_q.T], axis=1)
    msT = memory_slots.T
    wkvT = W_read_kv.T
    wm = write_matter.reshape(1, _DIM)

    out, mem = pl.pallas_call(
        _memory_kernel,
        grid=(_B, _S),
        in_specs=[
            pl.BlockSpec((1, 1, _L, _DIM), lambda b, j: (b, _S - 1 - j, 0, 0)),
            pl.BlockSpec((_DIM, 3 * _DIM), lambda b, j: (0, 0)),
            pl.BlockSpec((_DIM, _S), lambda b, j: (0, 0)),
            pl.BlockSpec((_DIM, _DIM), lambda b, j: (0, 0)),
            pl.BlockSpec((_DIM, 2 * _DIM), lambda b, j: (0, 0)),
            pl.BlockSpec((1, _DIM), lambda b, j: (0, 0)),
        ],
        out_specs=[
            pl.BlockSpec((1, 1, _L, _DIM), lambda b, j: (b, _S - 1 - j, 0, 0)),
            pl.BlockSpec((1, 1, 1, _DIM), lambda b, j: (b, _S - 1 - j, 0, 0)),
        ],
        out_shape=[
            jax.ShapeDtypeStruct((_B, _S, _L, _DIM), jnp.float32),
            jax.ShapeDtypeStruct((_B, _S, 1, _DIM), jnp.float32),
        ],
        scratch_shapes=[pltpu.VMEM((_S, 1, _DIM), jnp.float32)],
        compiler_params=pltpu.CompilerParams(
            dimension_semantics=("parallel", "arbitrary"),
            vmem_limit_bytes=100 * 1024 * 1024,
        ),
        name="memory_fused",
        interpret=interpret,
    )(xr, wcat, msT, W_write_q, wkvT, wm)
    return out.reshape(_B, _T, _DIM), mem.reshape(_B, _S, _DIM)


@jax.jit
def kernel(x, memory_slots, W_read_q, W_read_kv, W_write_q, W_write_k,
           W_write_v, write_matter):
    return _run(x, memory_slots, W_read_q, W_read_kv, W_write_q, W_write_k,
                W_write_v, write_matter)


# speedup vs baseline: 1.4952x; 1.4952x over previous
import math

import jax
import jax.numpy as jnp
from jax.experimental import pallas as pl
from jax.experimental.pallas import tpu as pltpu

_B, _T, _DIM = 4, 4096, 1024
_S, _H = 8, 8
_HD = _DIM // _H
_L = _T // _S
_NJ = 4
_SPS = _S // _NJ
_RPS = _SPS * _L
_C = _S * _H


def _memory_kernel(x_ref, wvq_ref, wwkT_ref, wwq_ref, msT_ref, wkv_ref, wm_ref,
                   out_ref, memout_ref, mem_scr, wscore_scr, v_scr, qr_scr):
    j = pl.program_id(1)
    base = (_NJ - 1 - j) * _SPS
    bf = jnp.bfloat16

    @pl.when(j == 0)
    def _():
        mem_scr[...] = jnp.zeros_like(mem_scr)
        qwT = jnp.dot(wwq_ref[...], msT_ref[...],
                      preferred_element_type=jnp.float32)
        ci = jax.lax.broadcasted_iota(jnp.int32, (_DIM, _C), 1)
        ri = jax.lax.broadcasted_iota(jnp.int32, (_DIM, _C), 0)
        sel = (ci % _H) == (ri // _HD)
        g = jnp.zeros((_DIM, _C), jnp.float32)
        for s in range(_S):
            g = g + jnp.where(sel & (ci // _H == s),
                              qwT[:, s:s + 1] * (1.0 / math.sqrt(_HD)), 0.0)
        wscore_scr[...] = jnp.dot(wwkT_ref[...], g.astype(bf),
                                  preferred_element_type=jnp.float32).astype(bf)

    xb = x_ref[0, 0].astype(bf)

    v_scr[...] = jnp.dot(xb, wvq_ref[..., :_DIM],
                         preferred_element_type=jnp.float32).astype(bf)
    qr_scr[...] = jnp.dot(xb, wvq_ref[..., _DIM:],
                          preferred_element_type=jnp.float32).astype(bf)

    scores = jnp.dot(xb, wscore_scr[...],
                     preferred_element_type=jnp.float32)
    ti = jax.lax.broadcasted_iota(jnp.int32, (_RPS, _C), 0) // _L
    ci2 = jax.lax.broadcasted_iota(jnp.int32, (_RPS, _C), 1) // _H
    wmask = ci2 == (ti + base)
    scores = jnp.where(wmask, scores, -1e30)
    mw = jnp.max(scores, axis=0, keepdims=True)
    ew = jnp.exp(scores - mw)
    attn_w = ew / (jnp.sum(ew, axis=0, keepdims=True) + 1e-30)
    attn_w = jnp.where(wmask, attn_w, 0.0).astype(bf)

    mem2 = jax.lax.dot_general(attn_w, v_scr[...], (((0,), (0,)), ((), ())),
                               preferred_element_type=jnp.float32)
    mem2r = mem2.reshape(_S, _H, _DIM)
    hmask = (jax.lax.broadcasted_iota(jnp.int32, (1, _H, _DIM), 2) // _HD
             == jax.lax.broadcasted_iota(jnp.int32, (1, _H, _DIM), 1))
    mem_new = jnp.sum(jnp.where(hmask, mem2r, 0.0), axis=1)
    mem_new = mem_new * wm_ref[...]
    var = jnp.sum(mem_new * mem_new, axis=1, keepdims=True) * (1.0 / _DIM)
    mem_new = mem_new * jax.lax.rsqrt(var + 1e-6)
    mem_scr[...] = mem_scr[...] + mem_new.reshape(_S, 1, _DIM)
    memout_ref[0, :, 0, :] = mem_scr[...].reshape(_S, _DIM)

    mem_all = mem_scr[...].reshape(_S, _DIM).astype(bf)
    kvr = jnp.dot(mem_all, wkv_ref[...], preferred_element_type=jnp.float32)
    kr = kvr[:, :_DIM].astype(bf)
    vr = kvr[:, _DIM:]

    qk = jax.lax.dot_general(qr_scr[...], kr, (((1,), (1,)), ((), ())),
                             preferred_element_type=jnp.float32)
    qk = qk * (1.0 / math.sqrt(_DIM))
    si = jax.lax.broadcasted_iota(jnp.int32, (_RPS, _S), 1)
    ti2 = jax.lax.broadcasted_iota(jnp.int32, (_RPS, _S), 0) // _L
    qk = jnp.where(si >= ti2 + base, qk, -1e30)
    mr = jnp.max(qk, axis=1, keepdims=True)
    er = jnp.exp(qk - mr)
    attn_r = er / jnp.sum(er, axis=1, keepdims=True)
    out = jnp.dot(attn_r, vr, preferred_element_type=jnp.float32)
    out_ref[0, 0] = out * jax.nn.sigmoid(out)


def _run(x, memory_slots, W_read_q, W_read_kv, W_write_q, W_write_k, W_write_v,
         write_matter, interpret=False):
    xr = x.reshape(_B, _NJ, _RPS, _DIM)
    bf = jnp.bfloat16
    wvq = jnp.concatenate([W_write_v.T, W_read_q.T], axis=1).astype(bf)
    wwkT = W_write_k.T.astype(bf)
    wwq = W_write_q.astype(bf)
    msT = memory_slots.T.astype(bf)
    wkvT = W_read_kv.T.astype(bf)
    wm = write_matter.reshape(1, _DIM)

    out, mem = pl.pallas_call(
        _memory_kernel,
        grid=(_B, _NJ),
        in_specs=[
            pl.BlockSpec((1, 1, _RPS, _DIM), lambda b, j: (b, _NJ - 1 - j, 0, 0)),
            pl.BlockSpec((_DIM, 2 * _DIM), lambda b, j: (0, 0)),
            pl.BlockSpec((_DIM, _DIM), lambda b, j: (0, 0)),
            pl.BlockSpec((_DIM, _DIM), lambda b, j: (0, 0)),
            pl.BlockSpec((_DIM, _S), lambda b, j: (0, 0)),
            pl.BlockSpec((_DIM, 2 * _DIM), lambda b, j: (0, 0)),
            pl.BlockSpec((1, _DIM), lambda b, j: (0, 0)),
        ],
        out_specs=[
            pl.BlockSpec((1, 1, _RPS, _DIM), lambda b, j: (b, _NJ - 1 - j, 0, 0)),
            pl.BlockSpec((1, _S, 1, _DIM), lambda b, j: (b, 0, 0, 0)),
        ],
        out_shape=[
            jax.ShapeDtypeStruct((_B, _NJ, _RPS, _DIM), jnp.float32),
            jax.ShapeDtypeStruct((_B, _S, 1, _DIM), jnp.float32),
        ],
        scratch_shapes=[
            pltpu.VMEM((_S, 1, _DIM), jnp.float32),
            pltpu.VMEM((_DIM, _C), jnp.bfloat16),
            pltpu.VMEM((_RPS, _DIM), jnp.bfloat16),
            pltpu.VMEM((_RPS, _DIM), jnp.bfloat16),
        ],
        compiler_params=pltpu.CompilerParams(
            dimension_semantics=("parallel", "arbitrary"),
            vmem_limit_bytes=60000 * 1024,
        ),
        name="memory_fused",
        interpret=interpret,
    )(xr, wvq, wwkT, wwq, msT, wkvT, wm)
    return out.reshape(_B, _T, _DIM), mem.reshape(_B, _S, _DIM)


@jax.jit
def kernel(x, memory_slots, W_read_q, W_read_kv, W_write_q, W_write_k,
           W_write_v, write_matter):
    return _run(x, memory_slots, W_read_q, W_read_kv, W_write_q, W_write_k,
                W_write_v, write_matter)


# speedup vs baseline: 1.7030x; 1.1390x over previous
import math

import jax
import jax.numpy as jnp
from jax.experimental import pallas as pl
from jax.experimental.pallas import tpu as pltpu

_B, _T, _DIM = 4, 4096, 1024
_S, _H = 8, 8
_HD = _DIM // _H
_L = _T // _S
_NJ = 4
_SPS = _S // _NJ
_RPS = _SPS * _L
_C = _S * _H


def _memory_kernel(x_ref, wwkT_ref, wwq_ref, msT_ref, wwv_ref, wrqT_ref,
                   wrk_ref, wrv_ref, wm8_ref,
                   out_ref, memoutT_ref, memT_scr, wscore_scr, wqk_scr):
    j = pl.program_id(1)
    base = (_NJ - 1 - j) * _SPS
    bf = jnp.bfloat16

    @pl.when(j == 0)
    def _():
        memT_scr[...] = jnp.zeros_like(memT_scr)
        qwT = jnp.dot(wwq_ref[...], msT_ref[...],
                      preferred_element_type=jnp.float32)
        ci = jax.lax.broadcasted_iota(jnp.int32, (_DIM, _C), 1)
        ri = jax.lax.broadcasted_iota(jnp.int32, (_DIM, _C), 0)
        hsel = (ci // _S) == (ri // _HD)
        g = jnp.zeros((_DIM, _C), jnp.float32)
        for s in range(_S):
            g = g + jnp.where(hsel & (ci % _S == s),
                              qwT[:, s:s + 1] * (1.0 / math.sqrt(_HD)), 0.0)
        wscore_scr[...] = jnp.dot(wwkT_ref[...], g.astype(bf),
                                  preferred_element_type=jnp.float32).astype(bf)
        wqk_scr[...] = jnp.dot(wrqT_ref[...], wrk_ref[...],
                               preferred_element_type=jnp.float32).astype(bf)

    xb = x_ref[0, 0].astype(bf)

    scores = jnp.dot(xb, wscore_scr[...],
                     preferred_element_type=jnp.float32)
    ti = jax.lax.broadcasted_iota(jnp.int32, (_RPS, _C), 0) // _L
    ci2 = jax.lax.broadcasted_iota(jnp.int32, (_RPS, _C), 1) % _S
    wmask = ci2 == (ti + base)
    scores = jnp.where(wmask, scores, -1e30)
    mw = jnp.max(scores, axis=0, keepdims=True)
    ew = jnp.exp(scores - mw)
    attn_w = ew / (jnp.sum(ew, axis=0, keepdims=True) + 1e-30)
    attn_w = jnp.where(wmask, attn_w, 0.0).astype(bf)

    pooledT = jax.lax.dot_general(xb, attn_w, (((0,), (0,)), ((), ())),
                                  preferred_element_type=jnp.float32)
    mem2T = jnp.dot(wwv_ref[...], pooledT.astype(bf),
                    preferred_element_type=jnp.float32)
    row_head = jax.lax.broadcasted_iota(jnp.int32, (_DIM, _S), 0) // _HD
    memT_new = jnp.zeros((_DIM, _S), jnp.float32)
    for h in range(_H):
        memT_new = memT_new + jnp.where(row_head == h,
                                        mem2T[:, h * _S:(h + 1) * _S], 0.0)
    memT_new = memT_new * wm8_ref[...]
    var = jnp.sum(memT_new * memT_new, axis=0, keepdims=True) * (1.0 / _DIM)
    memT_new = memT_new * jax.lax.rsqrt(var + 1e-6)
    memT_scr[...] = memT_scr[...] + memT_new
    memoutT_ref[0] = memT_scr[...]

    memb = memT_scr[...].astype(bf)
    p = jnp.dot(wqk_scr[...], memb,
                preferred_element_type=jnp.float32)
    qk = jnp.dot(xb, p.astype(bf),
                 preferred_element_type=jnp.float32) * (1.0 / math.sqrt(_DIM))
    si = jax.lax.broadcasted_iota(jnp.int32, (_RPS, _S), 1)
    ti2 = jax.lax.broadcasted_iota(jnp.int32, (_RPS, _S), 0) // _L
    qk = jnp.where(si >= ti2 + base, qk, -1e30)
    mr = jnp.max(qk, axis=1, keepdims=True)
    er = jnp.exp(qk - mr)
    attn_r = (er / jnp.sum(er, axis=1, keepdims=True)).astype(bf)
    vrT = jnp.dot(wrv_ref[...], memb,
                  preferred_element_type=jnp.float32)
    out = jax.lax.dot_general(attn_r, vrT.astype(bf), (((1,), (1,)), ((), ())),
                              preferred_element_type=jnp.float32)
    out_ref[0, 0] = out * jax.nn.sigmoid(out)


def _run(x, memory_slots, W_read_q, W_read_kv, W_write_q, W_write_k, W_write_v,
         write_matter, interpret=False):
    xr = x.reshape(_B, _NJ, _RPS, _DIM)
    bf = jnp.bfloat16
    wwkT = W_write_k.T.astype(bf)
    wwq = W_write_q.astype(bf)
    msT = memory_slots.T.astype(bf)
    wwv = W_write_v.astype(bf)
    wrqT = W_read_q.T.astype(bf)
    wrk = W_read_kv[:_DIM].astype(bf)
    wrv = W_read_kv[_DIM:].astype(bf)
    wm8 = jnp.broadcast_to(write_matter[:, None], (_DIM, _S))

    out, memT = pl.pallas_call(
        _memory_kernel,
        grid=(_B, _NJ),
        in_specs=[
            pl.BlockSpec((1, 1, _RPS, _DIM), lambda b, j: (b, _NJ - 1 - j, 0, 0)),
            pl.BlockSpec((_DIM, _DIM), lambda b, j: (0, 0)),
            pl.BlockSpec((_DIM, _DIM), lambda b, j: (0, 0)),
            pl.BlockSpec((_DIM, _S), lambda b, j: (0, 0)),
            pl.BlockSpec((_DIM, _DIM), lambda b, j: (0, 0)),
            pl.BlockSpec((_DIM, _DIM), lambda b, j: (0, 0)),
            pl.BlockSpec((_DIM, _DIM), lambda b, j: (0, 0)),
            pl.BlockSpec((_DIM, _DIM), lambda b, j: (0, 0)),
            pl.BlockSpec((_DIM, _S), lambda b, j: (0, 0)),
        ],
        out_specs=[
            pl.BlockSpec((1, 1, _RPS, _DIM), lambda b, j: (b, _NJ - 1 - j, 0, 0)),
            pl.BlockSpec((1, _DIM, _S), lambda b, j: (b, 0, 0)),
        ],
        out_shape=[
            jax.ShapeDtypeStruct((_B, _NJ, _RPS, _DIM), jnp.float32),
            jax.ShapeDtypeStruct((_B, _DIM, _S), jnp.float32),
        ],
        scratch_shapes=[
            pltpu.VMEM((_DIM, _S), jnp.float32),
            pltpu.VMEM((_DIM, _C), jnp.bfloat16),
            pltpu.VMEM((_DIM, _DIM), jnp.bfloat16),
        ],
        compiler_params=pltpu.CompilerParams(
            dimension_semantics=("parallel", "arbitrary"),
            vmem_limit_bytes=60000 * 1024,
        ),
        name="memory_fused",
        interpret=interpret,
    )(xr, wwkT, wwq, msT, wwv, wrqT, wrk, wrv, wm8)
    return out.reshape(_B, _T, _DIM), memT.transpose(0, 2, 1)


@jax.jit
def kernel(x, memory_slots, W_read_q, W_read_kv, W_write_q, W_write_k,
           W_write_v, write_matter):
    return _run(x, memory_slots, W_read_q, W_read_kv, W_write_q, W_write_k,
                W_write_v, write_matter)


# speedup vs baseline: 2.1778x; 1.2788x over previous
import math

import jax
import jax.numpy as jnp
from jax.experimental import pallas as pl
from jax.experimental.pallas import tpu as pltpu

_B, _T, _DIM = 4, 4096, 1024
_S, _H = 8, 8
_HD = _DIM // _H
_L = _T // _S
_NJ = 2
_SPS = _S // _NJ
_RPS = _SPS * _L
_C = _S * _H


def _prep_kernel(wwkT_ref, wwq_ref, msT_ref, wrqT_ref, wrk_ref,
                 wscore_ref, wqk_ref):
    bf = jnp.bfloat16
    qwT = jnp.dot(wwq_ref[...], msT_ref[...],
                  preferred_element_type=jnp.float32)
    ci = jax.lax.broadcasted_iota(jnp.int32, (_DIM, _C), 1)
    ri = jax.lax.broadcasted_iota(jnp.int32, (_DIM, _C), 0)
    hsel = (ci // _S) == (ri // _HD)
    g = jnp.zeros((_DIM, _C), jnp.float32)
    for s in range(_S):
        g = g + jnp.where(hsel & (ci % _S == s),
                          qwT[:, s:s + 1] * (1.0 / math.sqrt(_HD)), 0.0)
    wscore_ref[...] = jnp.dot(wwkT_ref[...], g.astype(bf),
                              preferred_element_type=jnp.float32).astype(bf)
    wqk_ref[...] = jnp.dot(wrqT_ref[...], wrk_ref[...],
                           preferred_element_type=jnp.float32).astype(bf)


def _memory_kernel(x_ref, wscore_ref, wqk_ref, wwv_ref, wrvT_ref, wm8_ref,
                   out_ref, memoutT_ref, memT_scr):
    j = pl.program_id(1)
    base = (_NJ - 1 - j) * _SPS
    bf = jnp.bfloat16

    @pl.when(j == 0)
    def _():
        memT_scr[...] = jnp.zeros_like(memT_scr)

    xb = x_ref[0, 0].astype(bf)

    scores = jnp.dot(xb, wscore_ref[...],
                     preferred_element_type=jnp.float32)
    ti = jax.lax.broadcasted_iota(jnp.int32, (_RPS, _C), 0) // _L
    ci2 = jax.lax.broadcasted_iota(jnp.int32, (_RPS, _C), 1) % _S
    wmask = ci2 == (ti + base)
    scores = jnp.where(wmask, scores, -1e30)
    mw = jnp.max(scores, axis=0, keepdims=True)
    ew = jnp.exp(scores - mw)
    attn_w = ew / (jnp.sum(ew, axis=0, keepdims=True) + 1e-30)
    attn_w = jnp.where(wmask, attn_w, 0.0).astype(bf)

    pooled = jax.lax.dot_general(attn_w, xb, (((0,), (0,)), ((), ())),
                                 preferred_element_type=jnp.float32)
    mem2T = jax.lax.dot_general(wwv_ref[...], pooled.astype(bf),
                                (((1,), (1,)), ((), ())),
                                preferred_element_type=jnp.float32)
    row_head = jax.lax.broadcasted_iota(jnp.int32, (_DIM, _S), 0) // _HD
    memT_new = jnp.zeros((_DIM, _S), jnp.float32)
    for h in range(_H):
        memT_new = memT_new + jnp.where(row_head == h,
                                        mem2T[:, h * _S:(h + 1) * _S], 0.0)
    memT_new = memT_new * wm8_ref[...]
    var = jnp.sum(memT_new * memT_new, axis=0, keepdims=True) * (1.0 / _DIM)
    memT_new = memT_new * jax.lax.rsqrt(var + 1e-6)
    memT_scr[...] = memT_scr[...] + memT_new
    memoutT_ref[0] = memT_scr[...]

    memb = memT_scr[...].astype(bf)
    p = jnp.dot(wqk_ref[...], memb,
                preferred_element_type=jnp.float32)
    qk = jnp.dot(xb, p.astype(bf),
                 preferred_element_type=jnp.float32) * (1.0 / math.sqrt(_DIM))
    si = jax.lax.broadcasted_iota(jnp.int32, (_RPS, _S), 1)
    ti2 = jax.lax.broadcasted_iota(jnp.int32, (_RPS, _S), 0) // _L
    qk = jnp.where(si >= ti2 + base, qk, -1e30)
    mr = jnp.max(qk, axis=1, keepdims=True)
    er = jnp.exp(qk - mr)
    attn_r = (er / jnp.sum(er, axis=1, keepdims=True)).astype(bf)
    vr = jax.lax.dot_general(memb, wrvT_ref[...], (((0,), (0,)), ((), ())),
                             preferred_element_type=jnp.float32)
    out = jnp.dot(attn_r, vr.astype(bf),
                  preferred_element_type=jnp.float32)
    out_ref[0, 0] = out * jax.nn.sigmoid(out)


def _run(x, memory_slots, W_read_q, W_read_kv, W_write_q, W_write_k, W_write_v,
         write_matter, interpret=False):
    xr = x.reshape(_B, _NJ, _RPS, _DIM)
    bf = jnp.bfloat16
    wwkT = W_write_k.T.astype(bf)
    wwq = W_write_q.astype(bf)
    msT = memory_slots.T.astype(bf)
    wrqT = W_read_q.T.astype(bf)
    wrk = W_read_kv[:_DIM].astype(bf)
    wwv = W_write_v.astype(bf)
    wrvT = W_read_kv[_DIM:].T.astype(bf)
    wm8 = jnp.broadcast_to(write_matter[:, None], (_DIM, _S))

    wscore, wqk = pl.pallas_call(
        _prep_kernel,
        out_shape=[
            jax.ShapeDtypeStruct((_DIM, _C), bf),
            jax.ShapeDtypeStruct((_DIM, _DIM), bf),
        ],
        name="memory_prep",
        interpret=interpret,
    )(wwkT, wwq, msT, wrqT, wrk)

    out, memT = pl.pallas_call(
        _memory_kernel,
        grid=(_B, _NJ),
        in_specs=[
            pl.BlockSpec((1, 1, _RPS, _DIM), lambda b, j: (b, _NJ - 1 - j, 0, 0)),
            pl.BlockSpec((_DIM, _C), lambda b, j: (0, 0)),
            pl.BlockSpec((_DIM, _DIM), lambda b, j: (0, 0)),
            pl.BlockSpec((_DIM, _DIM), lambda b, j: (0, 0)),
            pl.BlockSpec((_DIM, _DIM), lambda b, j: (0, 0)),
            pl.BlockSpec((_DIM, _S), lambda b, j: (0, 0)),
        ],
        out_specs=[
            pl.BlockSpec((1, 1, _RPS, _DIM), lambda b, j: (b, _NJ - 1 - j, 0, 0)),
            pl.BlockSpec((1, _DIM, _S), lambda b, j: (b, 0, 0)),
        ],
        out_shape=[
            jax.ShapeDtypeStruct((_B, _NJ, _RPS, _DIM), jnp.float32),
            jax.ShapeDtypeStruct((_B, _DIM, _S), jnp.float32),
        ],
        scratch_shapes=[
            pltpu.VMEM((_DIM, _S), jnp.float32),
        ],
        compiler_params=pltpu.CompilerParams(
            dimension_semantics=("parallel", "arbitrary"),
            vmem_limit_bytes=60000 * 1024,
        ),
        name="memory_fused",
        interpret=interpret,
    )(xr, wscore, wqk, wwv, wrvT, wm8)
    return out.reshape(_B, _T, _DIM), memT.transpose(0, 2, 1)


@jax.jit
def kernel(x, memory_slots, W_read_q, W_read_kv, W_write_q, W_write_k,
           W_write_v, write_matter):
    return _run(x, memory_slots, W_read_q, W_read_kv, W_write_q, W_write_k,
                W_write_v, write_matter)


# speedup vs baseline: 2.4105x; 1.1069x over previous
import math

import jax
import jax.numpy as jnp
from jax.experimental import pallas as pl
from jax.experimental.pallas import tpu as pltpu

_B, _T, _DIM = 4, 4096, 1024
_S, _H = 8, 8
_HD = _DIM // _H
_L = _T // _S
_NJ = 2
_SPS = _S // _NJ
_RPS = _SPS * _L
_C = _S * _H


def _prep_kernel(wwk_ref, wwq_ref, ms_ref, wrq_ref, wrk_ref, wwv_ref,
                 wscore_ref, wqk_ref, wwv_out_ref):
    bf = jnp.bfloat16
    qwT = jax.lax.dot_general(wwq_ref[...].astype(bf), ms_ref[...].astype(bf),
                              (((1,), (1,)), ((), ())),
                              preferred_element_type=jnp.float32)
    ci = jax.lax.broadcasted_iota(jnp.int32, (_DIM, _C), 1)
    ri = jax.lax.broadcasted_iota(jnp.int32, (_DIM, _C), 0)
    hsel = (ci // _S) == (ri // _HD)
    g = jnp.zeros((_DIM, _C), jnp.float32)
    for s in range(_S):
        g = g + jnp.where(hsel & (ci % _S == s),
                          qwT[:, s:s + 1] * (1.0 / math.sqrt(_HD)), 0.0)
    wscore_ref[...] = jax.lax.dot_general(
        wwk_ref[...].astype(bf), g.astype(bf), (((0,), (0,)), ((), ())),
        preferred_element_type=jnp.float32).astype(bf)
    wqk_ref[...] = jax.lax.dot_general(
        wrq_ref[...].astype(bf), wrk_ref[...].astype(bf),
        (((0,), (0,)), ((), ())),
        preferred_element_type=jnp.float32).astype(bf)
    wwv_out_ref[...] = wwv_ref[...].astype(bf)


def _memory_kernel(x_ref, wscore_ref, wqk_ref, wwv_ref, wrvT_ref, wm8_ref,
                   out_ref, memoutT_ref, memT_scr):
    j = pl.program_id(1)
    base = (_NJ - 1 - j) * _SPS
    bf = jnp.bfloat16

    @pl.when(j == 0)
    def _():
        memT_scr[...] = jnp.zeros_like(memT_scr)

    xb = x_ref[0, 0].astype(bf)

    scores = jnp.dot(xb, wscore_ref[...],
                     preferred_element_type=jnp.float32)
    ti = jax.lax.broadcasted_iota(jnp.int32, (_RPS, _C), 0) // _L
    ci2 = jax.lax.broadcasted_iota(jnp.int32, (_RPS, _C), 1) % _S
    wmask = ci2 == (ti + base)
    scores = jnp.where(wmask, scores, -1e30)
    mw = jnp.max(scores, axis=0, keepdims=True)
    ew = jnp.exp(scores - mw)
    attn_w = ew / (jnp.sum(ew, axis=0, keepdims=True) + 1e-30)
    attn_w = jnp.where(wmask, attn_w, 0.0).astype(bf)

    pooled = jax.lax.dot_general(attn_w, xb, (((0,), (0,)), ((), ())),
                                 preferred_element_type=jnp.float32)
    mem2T = jax.lax.dot_general(wwv_ref[...], pooled.astype(bf),
                                (((1,), (1,)), ((), ())),
                                preferred_element_type=jnp.float32)
    row_head = jax.lax.broadcasted_iota(jnp.int32, (_DIM, _S), 0) // _HD
    memT_new = jnp.zeros((_DIM, _S), jnp.float32)
    for h in range(_H):
        memT_new = memT_new + jnp.where(row_head == h,
                                        mem2T[:, h * _S:(h + 1) * _S], 0.0)
    memT_new = memT_new * wm8_ref[...]
    var = jnp.sum(memT_new * memT_new, axis=0, keepdims=True) * (1.0 / _DIM)
    memT_new = memT_new * jax.lax.rsqrt(var + 1e-6)
    memT_scr[...] = memT_scr[...] + memT_new
    memoutT_ref[0] = memT_scr[...]

    memb = memT_scr[...].astype(bf)
    p = jnp.dot(wqk_ref[...], memb,
                preferred_element_type=jnp.float32)
    qk = jnp.dot(xb, p.astype(bf),
                 preferred_element_type=jnp.float32) * (1.0 / math.sqrt(_DIM))
    si = jax.lax.broadcasted_iota(jnp.int32, (_RPS, _S), 1)
    ti2 = jax.lax.broadcasted_iota(jnp.int32, (_RPS, _S), 0) // _L
    qk = jnp.where(si >= ti2 + base, qk, -1e30)
    mr = jnp.max(qk, axis=1, keepdims=True)
    er = jnp.exp(qk - mr)
    attn_r = (er / jnp.sum(er, axis=1, keepdims=True)).astype(bf)
    vr = jax.lax.dot_general(memb, wrvT_ref[...], (((0,), (0,)), ((), ())),
                             preferred_element_type=jnp.float32)
    out = jnp.dot(attn_r, vr.astype(bf),
                  preferred_element_type=jnp.float32)
    out_ref[0, 0] = out * jax.nn.sigmoid(out)


def _run(x, memory_slots, W_read_q, W_read_kv, W_write_q, W_write_k, W_write_v,
         write_matter, interpret=False):
    xr = x.reshape(_B, _NJ, _RPS, _DIM)
    bf = jnp.bfloat16
    wrvT = W_read_kv[_DIM:].T.astype(bf)
    wm8 = jnp.broadcast_to(write_matter[:, None], (_DIM, _S))

    wscore, wqk, wwv = pl.pallas_call(
        _prep_kernel,
        out_shape=[
            jax.ShapeDtypeStruct((_DIM, _C), bf),
            jax.ShapeDtypeStruct((_DIM, _DIM), bf),
            jax.ShapeDtypeStruct((_DIM, _DIM), bf),
        ],
        name="memory_prep",
        interpret=interpret,
    )(W_write_k, W_write_q, memory_slots, W_read_q, W_read_kv[:_DIM],
      W_write_v)

    out, memT = pl.pallas_call(
        _memory_kernel,
        grid=(_B, _NJ),
        in_specs=[
            pl.BlockSpec((1, 1, _RPS, _DIM), lambda b, j: (b, _NJ - 1 - j, 0, 0)),
            pl.BlockSpec((_DIM, _C), lambda b, j: (0, 0)),
            pl.BlockSpec((_DIM, _DIM), lambda b, j: (0, 0)),
            pl.BlockSpec((_DIM, _DIM), lambda b, j: (0, 0)),
            pl.BlockSpec((_DIM, _DIM), lambda b, j: (0, 0)),
            pl.BlockSpec((_DIM, _S), lambda b, j: (0, 0)),
        ],
        out_specs=[
            pl.BlockSpec((1, 1, _RPS, _DIM), lambda b, j: (b, _NJ - 1 - j, 0, 0)),
            pl.BlockSpec((1, _DIM, _S), lambda b, j: (b, 0, 0)),
        ],
        out_shape=[
            jax.ShapeDtypeStruct((_B, _NJ, _RPS, _DIM), jnp.float32),
            jax.ShapeDtypeStruct((_B, _DIM, _S), jnp.float32),
        ],
        scratch_shapes=[
            pltpu.VMEM((_DIM, _S), jnp.float32),
        ],
        compiler_params=pltpu.CompilerParams(
            dimension_semantics=("parallel", "arbitrary"),
            vmem_limit_bytes=60000 * 1024,
        ),
        name="memory_fused",
        interpret=interpret,
    )(xr, wscore, wqk, wwv, wrvT, wm8)
    return out.reshape(_B, _T, _DIM), memT.transpose(0, 2, 1)


@jax.jit
def kernel(x, memory_slots, W_read_q, W_read_kv, W_write_q, W_write_k,
           W_write_v, write_matter):
    return _run(x, memory_slots, W_read_q, W_read_kv, W_write_q, W_write_k,
                W_write_v, write_matter)
